# baseline (speedup 1.0000x reference)
import functools

import jax
import jax.numpy as jnp
import numpy as np
from jax import lax
from jax.experimental import pallas as pl
from jax.experimental.pallas import tpu as pltpu
from jax.experimental.pallas import tpu_sc as plsc

GY, GX, D = 1024, 1024, 32
N = 250000

NC, NS = 2, 16
NW = NC * NS
G = 16
CHUNK = 7824
LAST = N - (NW - 1) * CHUNK
NG_FULL = CHUNK // G
NG_LAST = LAST // G
NPAD = NW * CHUNK


def _basis_coeffs():
    hermite = np.array(
        [[2, -2, 1, 1], [-3, 3, -2, -1], [0, 0, 1, 0], [1, 0, 0, 0]],
        dtype=np.float64)
    catmull = np.array(
        [[0, 1, 0, 0], [0, 0, 1, 0], [-0.5, 0, 0.5, 0], [0, -0.5, 0, 0.5]],
        dtype=np.float64)
    A = (hermite @ catmull)[::-1]
    return [[float(A[m, k]) for m in range(4)] for k in range(4)]


_COEF = _basis_coeffs()


def _tap_weights(s):
    s2 = s * s
    s3 = s2 * s
    out = []
    for k in range(4):
        c0, c1, c2, c3 = _COEF[k]
        out.append(c0 + c1 * s + c2 * s2 + c3 * s3)
    return out


def _make_sc_kernel():
    mesh = plsc.VectorSubcoreMesh(core_axis_name="c", subcore_axis_name="s")

    @functools.partial(
        pl.kernel,
        mesh=mesh,
        out_type=jax.ShapeDtypeStruct((N * D,), jnp.float32),
        compiler_params=pltpu.CompilerParams(
            use_tc_tiling_on_sc=False, needs_layout_passes=False),
        scratch_types=[
            pltpu.VMEM((CHUNK,), jnp.float32),
            pltpu.VMEM((CHUNK,), jnp.float32),
            pltpu.VMEM((128,), jnp.int32),
            pltpu.VMEM((128,), jnp.int32),
            pltpu.VMEM((128,), jnp.int32),
            pltpu.VMEM((128,), jnp.int32),
            pltpu.VMEM((128,), jnp.int32),
            pltpu.VMEM((128,), jnp.int32),
            pltpu.VMEM((128, D), jnp.bfloat16),
            pltpu.VMEM((128, D), jnp.bfloat16),
            pltpu.VMEM((128, D), jnp.bfloat16),
            pltpu.VMEM((128, D), jnp.bfloat16),
            pltpu.VMEM((128, D), jnp.bfloat16),
            pltpu.VMEM((128, D), jnp.bfloat16),
            pltpu.VMEM((G * D,), jnp.float32),
            pltpu.VMEM((G * D,), jnp.float32),
            pltpu.VMEM((G * D,), jnp.float32),
            pltpu.SemaphoreType.DMA,
            pltpu.SemaphoreType.DMA,
            pltpu.SemaphoreType.DMA,
            pltpu.SemaphoreType.DMA,
            pltpu.SemaphoreType.DMA,
            pltpu.SemaphoreType.DMA,
        ],
    )
    def spline_kernel(xs_hbm, ys_hbm, table_hbm, out_hbm,
                      xs_v, ys_v,
                      ia0, ib0, ia1, ib1, ia2, ib2,
                      ba0, bb0, ba1, bb1, ba2, bb2,
                      ov0, ov1, ov2,
                      sg0, sg1, sg2, so0, so1, so2):
        wid = lax.axis_index("s") * NC + lax.axis_index("c")
        base = wid * CHUNK
        obase = wid * CHUNK * D
        pltpu.sync_copy(xs_hbm.at[pl.ds(base, CHUNK)], xs_v)
        pltpu.sync_copy(ys_hbm.at[pl.ds(base, CHUNK)], ys_v)
        ng = jnp.where(wid == NW - 1, NG_LAST, NG_FULL)

        def fire(g, ia, ib, ba, bb, sg):
            off = g * G
            x = xs_v[pl.ds(off, G)]
            y = ys_v[pl.ds(off, G)]
            ix = x.astype(jnp.int32)
            iy = y.astype(jnp.int32)
            xcol = [jnp.clip(ix - 1 + k, 0, GX - 1) for k in range(4)]
            yrow = [jnp.clip(iy - 1 + j, 0, GY - 1) * GX for j in range(4)]
            for j in range(2):
                for k in range(4):
                    ia[pl.ds((j * 4 + k) * G, G)] = yrow[j] + xcol[k]
            for j in range(2, 4):
                for k in range(4):
                    ib[pl.ds(((j - 2) * 4 + k) * G, G)] = yrow[j] + xcol[k]
            pltpu.async_copy(table_hbm.at[ia], ba, sg)
            pltpu.async_copy(table_hbm.at[ib], bb, sg)

        def compute(g, ia, ib, ba, bb, ov, sg, so, drain_out):
            off = g * G
            x = xs_v[pl.ds(off, G)]
            y = ys_v[pl.ds(off, G)]
            ix = x.astype(jnp.int32)
            iy = y.astype(jnp.int32)
            sx = jnp.clip(x - ix.astype(jnp.float32), 0.0, 1.0)
            sy = jnp.clip(y - iy.astype(jnp.float32), 0.0, 1.0)
            cx = _tap_weights(sx)
            cy = _tap_weights(sy)

            @pl.when(drain_out)
            def _():
                pltpu.make_async_copy(
                    ov, out_hbm.at[pl.ds(obase, G * D)], so).wait()

            pltpu.make_async_copy(table_hbm.at[ia], ba, sg).wait()
            pltpu.make_async_copy(table_hbm.at[ib], bb, sg).wait()
            iota2 = lax.iota(jnp.int32, 16) * 2
            for p in range(G):
                aw = [cx[k][p] for k in range(4)]
                bw = [cy[j][p] for j in range(4)]
                acc_e = None
                acc_o = None
                for j in range(4):
                    buf = ba if j < 2 else bb
                    jj = j % 2
                    se = None
                    so_ = None
                    for k in range(4):
                        r = (jj * 4 + k) * G + p
                        packed = plsc.bitcast(buf[r, :], jnp.int32)
                        ev = plsc.bitcast(packed << 16, jnp.float32)
                        od = plsc.bitcast(
                            packed & jnp.int32(-65536), jnp.float32)
                        te = aw[k] * ev
                        to = aw[k] * od
                        se = te if se is None else se + te
                        so_ = to if so_ is None else so_ + to
                    acc_e = bw[j] * se if acc_e is None else acc_e + bw[j] * se
                    acc_o = bw[j] * so_ if acc_o is None else acc_o + bw[j] * so_
                off_p = p * D
                plsc.store_scatter(ov, [iota2 + off_p], acc_e)
                plsc.store_scatter(ov, [iota2 + (off_p + 1)], acc_o)
            pltpu.async_copy(
                ov, out_hbm.at[pl.ds(obase + off * D, G * D)], so)

        slots = [
            (ia0, ib0, ba0, bb0, ov0, sg0, so0),
            (ia1, ib1, ba1, bb1, ov1, sg1, so1),
            (ia2, ib2, ba2, bb2, ov2, sg2, so2),
        ]

        def fire_slot(g, s):
            ia, ib, ba, bb, _, sg, _ = slots[s]
            fire(g, ia, ib, ba, bb, sg)

        def compute_slot(g, s, drain):
            ia, ib, ba, bb, ov, sg, so = slots[s]
            compute(g, ia, ib, ba, bb, ov, sg, so, drain)

        fire_slot(0, 0)
        fire_slot(1, 1)
        num_triples = (ng + 2) // 3

        def body(p, carry):
            g = 3 * p

            @pl.when(g + 2 < ng)
            def _():
                fire_slot(g + 2, 2)

            compute_slot(g, 0, p > 0)

            @pl.when(g + 3 < ng)
            def _():
                fire_slot(g + 3, 0)

            @pl.when(g + 1 < ng)
            def _():
                compute_slot(g + 1, 1, p > 0)

            @pl.when(g + 4 < ng)
            def _():
                fire_slot(g + 4, 1)

            @pl.when(g + 2 < ng)
            def _():
                compute_slot(g + 2, 2, p > 0)

            return carry

        lax.fori_loop(0, num_triples, body, 0)
        for ov, so in ((ov0, so0), (ov1, so1), (ov2, so2)):
            pltpu.make_async_copy(
                ov, out_hbm.at[pl.ds(obase, G * D)], so).wait()

    return spline_kernel


_SPLINE = _make_sc_kernel()


def kernel(pts, ControlPoints):
    xs = jnp.pad(pts[:, 0], (0, NPAD - N))
    ys = jnp.pad(pts[:, 1], (0, NPAD - N))
    table = ControlPoints.astype(jnp.bfloat16).reshape(GY * GX, D)
    return _SPLINE(xs, ys, table).reshape(N, D)


# speedup vs baseline: 1.1547x; 1.1547x over previous
import functools

import jax
import jax.numpy as jnp
import numpy as np
from jax import lax
from jax.experimental import pallas as pl
from jax.experimental.pallas import tpu as pltpu
from jax.experimental.pallas import tpu_sc as plsc

GY, GX, D = 1024, 1024, 32
N = 250000

NC, NS = 2, 16
NW = NC * NS
G = 16
CHUNK = 7824
LAST = N - (NW - 1) * CHUNK
NG_FULL = CHUNK // G
NG_LAST = LAST // G
NPAD = NW * CHUNK


def _basis_coeffs():
    hermite = np.array(
        [[2, -2, 1, 1], [-3, 3, -2, -1], [0, 0, 1, 0], [1, 0, 0, 0]],
        dtype=np.float64)
    catmull = np.array(
        [[0, 1, 0, 0], [0, 0, 1, 0], [-0.5, 0, 0.5, 0], [0, -0.5, 0, 0.5]],
        dtype=np.float64)
    A = (hermite @ catmull)[::-1]
    return [[float(A[m, k]) for m in range(4)] for k in range(4)]


_COEF = _basis_coeffs()


def _tap_weights(s):
    s2 = s * s
    s3 = s2 * s
    out = []
    for k in range(4):
        c0, c1, c2, c3 = _COEF[k]
        out.append(c0 + c1 * s + c2 * s2 + c3 * s3)
    return out


def _make_sc_kernel():
    mesh = plsc.VectorSubcoreMesh(core_axis_name="c", subcore_axis_name="s")

    @functools.partial(
        pl.kernel,
        mesh=mesh,
        out_type=jax.ShapeDtypeStruct((N * D,), jnp.float32),
        compiler_params=pltpu.CompilerParams(
            use_tc_tiling_on_sc=False, needs_layout_passes=False),
        scratch_types=[
            pltpu.VMEM((CHUNK,), jnp.float32),
            pltpu.VMEM((CHUNK,), jnp.float32),
            pltpu.VMEM((128,), jnp.int32),
            pltpu.VMEM((128,), jnp.int32),
            pltpu.VMEM((128,), jnp.int32),
            pltpu.VMEM((128,), jnp.int32),
            pltpu.VMEM((128,), jnp.int32),
            pltpu.VMEM((128,), jnp.int32),
            pltpu.VMEM((128, D), jnp.float32),
            pltpu.VMEM((128, D), jnp.float32),
            pltpu.VMEM((128, D), jnp.float32),
            pltpu.VMEM((128, D), jnp.float32),
            pltpu.VMEM((128, D), jnp.float32),
            pltpu.VMEM((128, D), jnp.float32),
            pltpu.VMEM((G * D,), jnp.float32),
            pltpu.VMEM((G * D,), jnp.float32),
            pltpu.VMEM((G * D,), jnp.float32),
            pltpu.SemaphoreType.DMA,
            pltpu.SemaphoreType.DMA,
            pltpu.SemaphoreType.DMA,
            pltpu.SemaphoreType.DMA,
            pltpu.SemaphoreType.DMA,
            pltpu.SemaphoreType.DMA,
        ],
    )
    def spline_kernel(xs_hbm, ys_hbm, table_hbm, out_hbm,
                      xs_v, ys_v,
                      ia0, ib0, ia1, ib1, ia2, ib2,
                      ba0, bb0, ba1, bb1, ba2, bb2,
                      ov0, ov1, ov2,
                      sg0, sg1, sg2, so0, so1, so2):
        wid = lax.axis_index("s") * NC + lax.axis_index("c")
        base = wid * CHUNK
        obase = wid * CHUNK * D
        pltpu.sync_copy(xs_hbm.at[pl.ds(base, CHUNK)], xs_v)
        pltpu.sync_copy(ys_hbm.at[pl.ds(base, CHUNK)], ys_v)
        ng = jnp.where(wid == NW - 1, NG_LAST, NG_FULL)

        def fire(g, ia, ib, ba, bb, sg):
            off = g * G
            x = xs_v[pl.ds(off, G)]
            y = ys_v[pl.ds(off, G)]
            ix = x.astype(jnp.int32)
            iy = y.astype(jnp.int32)
            xcol = [jnp.clip(ix - 1 + k, 0, GX - 1) for k in range(4)]
            yrow = [jnp.clip(iy - 1 + j, 0, GY - 1) * GX for j in range(4)]
            for j in range(2):
                for k in range(4):
                    ia[pl.ds((j * 4 + k) * G, G)] = yrow[j] + xcol[k]
            for j in range(2, 4):
                for k in range(4):
                    ib[pl.ds(((j - 2) * 4 + k) * G, G)] = yrow[j] + xcol[k]
            pltpu.async_copy(table_hbm.at[ia], ba, sg)
            pltpu.async_copy(table_hbm.at[ib], bb, sg)

        def compute(g, ia, ib, ba, bb, ov, sg, so, drain_out):
            off = g * G
            x = xs_v[pl.ds(off, G)]
            y = ys_v[pl.ds(off, G)]
            ix = x.astype(jnp.int32)
            iy = y.astype(jnp.int32)
            sx = jnp.clip(x - ix.astype(jnp.float32), 0.0, 1.0)
            sy = jnp.clip(y - iy.astype(jnp.float32), 0.0, 1.0)
            cx = _tap_weights(sx)
            cy = _tap_weights(sy)

            @pl.when(drain_out)
            def _():
                pltpu.make_async_copy(
                    ov, out_hbm.at[pl.ds(obase, G * D)], so).wait()

            pltpu.make_async_copy(table_hbm.at[ia], ba, sg).wait()
            pltpu.make_async_copy(table_hbm.at[ib], bb, sg).wait()
            for p in range(G):
                aw = [cx[k][p] for k in range(4)]
                bw = [cy[j][p] for j in range(4)]
                for h in range(2):
                    acc = None
                    for j in range(4):
                        buf = ba if j < 2 else bb
                        jj = j % 2
                        srow = None
                        for k in range(4):
                            r = (jj * 4 + k) * G + p
                            row = buf[r, pl.ds(h * 16, 16)]
                            term = aw[k] * row
                            srow = term if srow is None else srow + term
                        term2 = bw[j] * srow
                        acc = term2 if acc is None else acc + term2
                    ov[pl.ds(p * D + h * 16, 16)] = acc
            pltpu.async_copy(
                ov, out_hbm.at[pl.ds(obase + off * D, G * D)], so)

        slots = [
            (ia0, ib0, ba0, bb0, ov0, sg0, so0),
            (ia1, ib1, ba1, bb1, ov1, sg1, so1),
            (ia2, ib2, ba2, bb2, ov2, sg2, so2),
        ]

        def fire_slot(g, s):
            ia, ib, ba, bb, _, sg, _ = slots[s]
            fire(g, ia, ib, ba, bb, sg)

        def compute_slot(g, s, drain):
            ia, ib, ba, bb, ov, sg, so = slots[s]
            compute(g, ia, ib, ba, bb, ov, sg, so, drain)

        fire_slot(0, 0)
        fire_slot(1, 1)
        num_triples = (ng + 2) // 3

        def body(p, carry):
            g = 3 * p

            @pl.when(g + 2 < ng)
            def _():
                fire_slot(g + 2, 2)

            compute_slot(g, 0, p > 0)

            @pl.when(g + 3 < ng)
            def _():
                fire_slot(g + 3, 0)

            @pl.when(g + 1 < ng)
            def _():
                compute_slot(g + 1, 1, p > 0)

            @pl.when(g + 4 < ng)
            def _():
                fire_slot(g + 4, 1)

            @pl.when(g + 2 < ng)
            def _():
                compute_slot(g + 2, 2, p > 0)

            return carry

        lax.fori_loop(0, num_triples, body, 0)
        for ov, so in ((ov0, so0), (ov1, so1), (ov2, so2)):
            pltpu.make_async_copy(
                ov, out_hbm.at[pl.ds(obase, G * D)], so).wait()

    return spline_kernel


_SPLINE = _make_sc_kernel()


def kernel(pts, ControlPoints):
    xs = jnp.pad(pts[:, 0], (0, NPAD - N))
    ys = jnp.pad(pts[:, 1], (0, NPAD - N))
    table = ControlPoints.reshape(GY * GX, D)
    return _SPLINE(xs, ys, table).reshape(N, D)
